# baseline (speedup 1.0000x reference)
import functools

import jax
import jax.numpy as jnp
from jax import lax
from jax.experimental import pallas as pl
from jax.experimental.pallas import tpu as pltpu
from jax.experimental.pallas import tpu_sc as plsc

N_NODES = 50000
IN_CHS = 128
HID = 64
N_CLUSTERS = 2500

NC = 2
NS = 16
LANES = 16
NW = NC * NS

CPT = 80
C_PAD = NW * CPT
TC_TILE = 2000
TC_GRID = N_NODES // TC_TILE
N_PAD = 51200
SEG_CHUNK = 512
SEG_CHUNK2 = 384
G2 = 128


@functools.lru_cache(maxsize=None)
def _sc_mesh():
    return plsc.VectorSubcoreMesh(core_axis_name="c", subcore_axis_name="s")


def _ln_tail(hm, g, b):
    ones = jnp.full((HID, HID), 1.0 / HID, jnp.float32)
    d = hm[:, :HID] - hm[:, HID:]
    v = jnp.dot(d * d, ones)
    return jax.nn.relu(d * lax.rsqrt(v + 1e-5) * g + b)


def _mlp0_body(x_ref, w1_ref, b1_ref, g1_ref, e1_ref, w2_ref, b2_ref,
               g2_ref, e2_ref, wnext_ref, oh_ref, oa_ref):
    hm = jnp.dot(x_ref[...], w1_ref[...]) + b1_ref[...]
    h = _ln_tail(hm, g1_ref[...], e1_ref[...])
    hm2 = jnp.dot(h, w2_ref[...]) + b2_ref[...]
    h2 = _ln_tail(hm2, g2_ref[...], e2_ref[...])
    oh_ref[...] = h2
    oa_ref[...] = jnp.dot(h2, wnext_ref[...])


def _mlp_cat_body(a_ref, g_ref, w1b_ref, b1_ref, g1_ref, e1_ref,
                  w2_ref, b2_ref, g2_ref, e2_ref, wnext_ref, oh_ref, oa_ref):
    hm = a_ref[...] + jnp.dot(g_ref[...], w1b_ref[...]) + b1_ref[...]
    h = _ln_tail(hm, g1_ref[...], e1_ref[...])
    hm2 = jnp.dot(h, w2_ref[...]) + b2_ref[...]
    h2 = _ln_tail(hm2, g2_ref[...], e2_ref[...])
    oh_ref[...] = h2
    oa_ref[...] = jnp.dot(h2, wnext_ref[...])


def _mlp_cat_ext_body(a_ref, g_ref, w1b_ref, b1_ref, g1_ref, e1_ref,
                      w2_ref, b2_ref, g2_ref, e2_ref, wlin_ref, o_ref):
    hm = a_ref[...] + jnp.dot(g_ref[...], w1b_ref[...]) + b1_ref[...]
    h = _ln_tail(hm, g1_ref[...], e1_ref[...])
    hm2 = jnp.dot(h, w2_ref[...]) + b2_ref[...]
    h2 = _ln_tail(hm2, g2_ref[...], e2_ref[...])
    y2 = jnp.dot(h2, wlin_ref[...])
    o_ref[...] = jnp.concatenate([h2, y2], axis=1)


def _final_body(a_ref, wb_ref, b_ref, o_ref):
    a = a_ref[...][:, :HID]
    y = a_ref[...][:, HID:]
    pre = y + jnp.dot(a, wb_ref[...]) + b_ref[...]
    n = jnp.sqrt(jnp.sum(pre * pre, axis=-1, keepdims=True))
    o_ref[...] = pre / jnp.maximum(n, 1e-12)


def _aug_w(W):
    m = jnp.broadcast_to(jnp.mean(W, axis=1, keepdims=True), W.shape)
    return jnp.concatenate([W, m], axis=1)


def _aug_b(b):
    return jnp.concatenate([b, jnp.broadcast_to(jnp.mean(b), b.shape)])


def _row_spec(width):
    return pl.BlockSpec((TC_TILE, width), lambda i: (i, 0))


def _full_spec(r, c):
    return pl.BlockSpec((r, c), lambda i: (0, 0))


def _vec_spec(n=HID):
    return pl.BlockSpec((n,), lambda i: (0,))


def _mlp0(x, w1, b1, g1, e1, w2, b2, g2, e2, wnext):
    return pl.pallas_call(
        _mlp0_body,
        grid=(TC_GRID,),
        in_specs=[_row_spec(IN_CHS), _full_spec(IN_CHS, 2 * HID),
                  _vec_spec(2 * HID), _vec_spec(), _vec_spec(),
                  _full_spec(HID, 2 * HID), _vec_spec(2 * HID), _vec_spec(),
                  _vec_spec(), _full_spec(HID, 2 * HID)],
        out_specs=[_row_spec(HID), _row_spec(2 * HID)],
        out_shape=[jax.ShapeDtypeStruct((N_PAD, HID), jnp.float32),
                   jax.ShapeDtypeStruct((N_PAD, 2 * HID), jnp.float32)],
    )(x, _aug_w(w1), _aug_b(b1), g1, e1, _aug_w(w2), _aug_b(b2), g2, e2,
      _aug_w(wnext))


def _mlp_cat(a, g, w1b, b1, g1, e1, w2, b2, g2, e2, wnext):
    return pl.pallas_call(
        _mlp_cat_body,
        grid=(TC_GRID,),
        in_specs=[_row_spec(2 * HID), _row_spec(HID),
                  _full_spec(HID, 2 * HID), _vec_spec(2 * HID), _vec_spec(),
                  _vec_spec(), _full_spec(HID, 2 * HID), _vec_spec(2 * HID),
                  _vec_spec(), _vec_spec(), _full_spec(HID, 2 * HID)],
        out_specs=[_row_spec(HID), _row_spec(2 * HID)],
        out_shape=[jax.ShapeDtypeStruct((N_PAD, HID), jnp.float32),
                   jax.ShapeDtypeStruct((N_PAD, 2 * HID), jnp.float32)],
    )(a, g, _aug_w(w1b), _aug_b(b1), g1, e1, _aug_w(w2), _aug_b(b2), g2, e2,
      _aug_w(wnext))


def _mlp_cat_ext(a, g, w1b, b1, g1, e1, w2, b2, g2, e2, wlin):
    return pl.pallas_call(
        _mlp_cat_ext_body,
        grid=(TC_GRID,),
        in_specs=[_row_spec(2 * HID), _row_spec(HID),
                  _full_spec(HID, 2 * HID), _vec_spec(2 * HID), _vec_spec(),
                  _vec_spec(), _full_spec(HID, 2 * HID), _vec_spec(2 * HID),
                  _vec_spec(), _vec_spec(), _full_spec(HID, HID)],
        out_specs=_row_spec(2 * HID),
        out_shape=jax.ShapeDtypeStruct((N_PAD, 2 * HID), jnp.float32),
    )(a, g, _aug_w(w1b), _aug_b(b1), g1, e1, _aug_w(w2), _aug_b(b2), g2, e2,
      wlin)


def _final(agg2, wb, b):
    return pl.pallas_call(
        _final_body,
        in_specs=[pl.BlockSpec((C_PAD, 2 * HID), lambda: (0, 0)),
                  pl.BlockSpec((HID, HID), lambda: (0, 0)),
                  pl.BlockSpec((HID,), lambda: (0,))],
        out_specs=pl.BlockSpec((C_PAD, HID), lambda: (0, 0)),
        out_shape=jax.ShapeDtypeStruct((C_PAD, HID), jnp.float32),
    )(agg2, wb, b)


def _tile_prologue(starts_hbm, stbuf, accbuf, chunk, width):
    wid = lax.axis_index("s") * NC + lax.axis_index("c")
    lane = lax.iota(jnp.int32, LANES)

    pltpu.sync_copy(starts_hbm, stbuf)

    def _extract(i):
        return plsc.load_gather(stbuf, [jnp.full((LANES,), i)])[0]

    s0 = _extract(wid)
    s1 = _extract(wid + 1)
    cbase = wid * CPT

    neg = jnp.full((LANES,), -jnp.inf, jnp.float32)
    kw = width // LANES

    def _init(i, carry):
        accbuf[i // kw, pl.ds((i % kw) * LANES, LANES)] = neg
        return carry

    lax.fori_loop(0, CPT * kw, _init, 0)

    base = (s0 // 8) * 8
    total = s1 - base
    nchunks = (total + chunk - 1) // chunk
    return wid, lane, cbase, base, nchunks, neg, s0, s1


def _segmax_pass(h_hbm, cl_hbm, hbuf, clbuf, accbuf, lane, cbase, base,
                 nchunks, neg, semh, semh2, semc, chunk, width):
    kw = width // LANES

    half = chunk // 2

    def _start(ci, b):
        st = base + ci * chunk
        pltpu.async_copy(h_hbm.at[pl.ds(st, half), :],
                         hbuf.at[b].at[pl.ds(0, half)], semh[b])
        pltpu.async_copy(h_hbm.at[pl.ds(st + half, half), :],
                         hbuf.at[b].at[pl.ds(half, half)], semh2[b])
        pltpu.async_copy(cl_hbm.at[pl.ds(st, chunk)], clbuf.at[b], semc[b])

    def _wait(b):
        pltpu.make_async_copy(h_hbm.at[pl.ds(0, half), :],
                              hbuf.at[b].at[pl.ds(0, half)], semh[b]).wait()
        pltpu.make_async_copy(h_hbm.at[pl.ds(0, half), :],
                              hbuf.at[b].at[pl.ds(half, half)],
                              semh2[b]).wait()
        pltpu.make_async_copy(cl_hbm.at[pl.ds(0, chunk)], clbuf.at[b],
                              semc[b]).wait()

    cbase_v = jnp.full((LANES,), cbase)

    def _compute(b, carry):
        def _group(gi, carry2):
            prev_v, accs = carry2
            cids = clbuf[b, pl.ds(gi * LANES, LANES)]
            for k in range(LANES):
                j = gi * LANES + k
                cid_v = cids.at[jnp.full((LANES,), k, jnp.int32)].get(
                    mode="promise_in_bounds")
                same_v = cid_v == prev_v
                c_loc = cid_v - cbase_v
                valid_v = (c_loc >= 0) & (c_loc < CPT)
                row = jnp.where(valid_v, c_loc, 0)
                new_accs = []
                for w in range(kw):
                    r = hbuf[b, j, pl.ds(w * LANES, LANES)]
                    a = jnp.where(same_v, jnp.maximum(accs[w], r), r)
                    plsc.store_scatter(accbuf, [row, lane + w * LANES], a,
                                       mask=valid_v)
                    new_accs.append(a)
                accs = tuple(new_accs)
                prev_v = cid_v
            return (prev_v, accs)

        return lax.fori_loop(0, chunk // LANES, _group, carry)

    _start(0, 0)
    init = (jnp.full((LANES,), -1, jnp.int32), (neg,) * kw)

    def _pair(pi, carry):
        for b in range(2):
            ci = pi * 2 + b

            def _proc(c, ci=ci, b=b):
                _start(ci + 1, 1 - b)
                _wait(b)
                return _compute(b, c)

            carry = lax.cond(ci < nchunks, _proc, lambda c: c, carry)
        return carry

    lax.fori_loop(0, (nchunks + 1) // 2, _pair, init)

    @pl.when(nchunks % 2 == 0)
    def _():
        _wait(0)

    @pl.when(nchunks % 2 == 1)
    def _():
        _wait(1)


def _segmax2_sc_body(h_hbm, cl_hbm, starts_hbm, agg_hbm, hbuf, clbuf, accbuf,
                     stbuf, semh0, semh1, semg0, semg1, semc0, semc1):
    wid, lane, cbase, base, nchunks, neg, s0, s1 = _tile_prologue(
        starts_hbm, stbuf, accbuf, SEG_CHUNK2, 2 * HID)
    _segmax_pass(h_hbm, cl_hbm, hbuf, clbuf, accbuf, lane, cbase, base,
                 nchunks, neg, (semh0, semh1), (semg0, semg1), (semc0, semc1),
                 SEG_CHUNK2, 2 * HID)
    pltpu.sync_copy(accbuf, agg_hbm.at[pl.ds(cbase, CPT), :])


def _segmax2(h_pad, cl_pad, starts):
    return pl.kernel(
        _segmax2_sc_body,
        out_type=jax.ShapeDtypeStruct((C_PAD, 2 * HID), jnp.float32),
        mesh=_sc_mesh(),
        compiler_params=pltpu.CompilerParams(needs_layout_passes=False,
                                             use_tc_tiling_on_sc=False),
        scratch_types=[
            pltpu.VMEM((2, SEG_CHUNK2, 2 * HID), jnp.float32),
            pltpu.VMEM((2, SEG_CHUNK2), jnp.int32),
            pltpu.VMEM((CPT, 2 * HID), jnp.float32),
            pltpu.VMEM((48,), jnp.int32),
            pltpu.SemaphoreType.DMA,
            pltpu.SemaphoreType.DMA,
            pltpu.SemaphoreType.DMA,
            pltpu.SemaphoreType.DMA,
            pltpu.SemaphoreType.DMA,
            pltpu.SemaphoreType.DMA,
        ],
    )(h_pad, cl_pad, starts)


def _segmax_gather_sc_body(h_hbm, cl_hbm, starts_hbm, g_hbm, hbuf, clbuf,
                           accbuf, stbuf, cl2buf, idxbuf, gbuf,
                           semh0, semh1, semg0, semg1, semc0, semc1,
                           semi0, semi1, semo0, semo1):
    wid, lane, cbase, base, nchunks, neg, s0, s1 = _tile_prologue(
        starts_hbm, stbuf, accbuf, SEG_CHUNK, HID)
    _segmax_pass(h_hbm, cl_hbm, hbuf, clbuf, accbuf, lane, cbase, base,
                 nchunks, neg, (semh0, semh1), (semg0, semg1), (semc0, semc1),
                 SEG_CHUNK, HID)

    semi = (semi0, semi1)
    semo = (semo0, semo1)
    dummy = jnp.int32(N_NODES) + wid
    nch2 = nchunks * (SEG_CHUNK // G2)

    def _start2(ci, b):
        st = base + ci * G2
        pltpu.async_copy(cl_hbm.at[pl.ds(st, G2)], cl2buf.at[b], semi[b])

    def _wait2(b):
        pltpu.make_async_copy(cl_hbm.at[pl.ds(0, G2)], cl2buf.at[b],
                              semi[b]).wait()

    def _wait_out(b):
        pltpu.make_async_copy(gbuf.at[b], g_hbm.at[pl.ds(0, G2), :],
                              semo[b]).wait()

    cbase_v2 = jnp.full((LANES,), cbase)

    def _compute2(ci, b):
        start = base + ci * G2

        def _g2(gi, carry3):
            cids = cl2buf[b, pl.ds(gi * LANES, LANES)]
            for k in range(LANES):
                j = gi * LANES + k
                cid_v = cids.at[jnp.full((LANES,), k, jnp.int32)].get(
                    mode="promise_in_bounds")
                csp = jnp.clip(cid_v - cbase_v2, 0, CPT - 1)
                gbuf[b, j, pl.ds(0, LANES)] = plsc.load_gather(
                    accbuf, [csp, lane])
                gbuf[b, j, pl.ds(LANES, LANES)] = plsc.load_gather(
                    accbuf, [csp, lane + LANES])
                gbuf[b, j, pl.ds(2 * LANES, LANES)] = plsc.load_gather(
                    accbuf, [csp, lane + 2 * LANES])
                gbuf[b, j, pl.ds(3 * LANES, LANES)] = plsc.load_gather(
                    accbuf, [csp, lane + 3 * LANES])
            return carry3

        lax.fori_loop(0, G2 // LANES, _g2, 0)
        interior = (start >= s0) & (start + G2 <= s1)

        @pl.when(interior)
        def _():
            pltpu.async_copy(gbuf.at[b], g_hbm.at[pl.ds(start, G2), :],
                             semo[b])

        @pl.when(jnp.logical_not(interior))
        def _():
            for grp in range(G2 // LANES):
                cids = cl2buf[b, pl.ds(grp * LANES, LANES)]
                c_loc = cids - cbase
                valid = (c_loc >= 0) & (c_loc < CPT)
                node_v = jnp.full((LANES,), start + grp * LANES) + lane
                idxbuf[b, pl.ds(grp * LANES, LANES)] = jnp.where(
                    valid, node_v, jnp.full((LANES,), dummy))
            pltpu.async_copy(gbuf.at[b], g_hbm.at[idxbuf.at[b]], semo[b])

    _start2(0, 0)

    def _pair2(pi, carry):
        for b in range(2):
            ci = pi * 2 + b

            @pl.when(ci < nch2)
            def _(ci=ci, b=b):
                _start2(ci + 1, 1 - b)
                _wait2(b)

                @pl.when(ci >= 2)
                def _():
                    _wait_out(b)

                _compute2(ci, b)
        return carry

    lax.fori_loop(0, (nch2 + 1) // 2, _pair2, 0)

    @pl.when(nch2 % 2 == 0)
    def _():
        _wait2(0)

    @pl.when(nch2 % 2 == 1)
    def _():
        _wait2(1)

    @pl.when(nch2 >= 1)
    def _():
        _wait_out(0)

    @pl.when(nch2 >= 2)
    def _():
        _wait_out(1)


def _segmax_gather(h_pad, cl_pad, starts):
    return pl.kernel(
        _segmax_gather_sc_body,
        out_type=jax.ShapeDtypeStruct((N_PAD, HID), jnp.float32),
        mesh=_sc_mesh(),
        compiler_params=pltpu.CompilerParams(needs_layout_passes=False,
                                             use_tc_tiling_on_sc=False),
        scratch_types=[
            pltpu.VMEM((2, SEG_CHUNK, HID), jnp.float32),
            pltpu.VMEM((2, SEG_CHUNK), jnp.int32),
            pltpu.VMEM((CPT, HID), jnp.float32),
            pltpu.VMEM((48,), jnp.int32),
            pltpu.VMEM((2, G2), jnp.int32),
            pltpu.VMEM((2, G2), jnp.int32),
            pltpu.VMEM((2, G2, HID), jnp.float32),
            pltpu.SemaphoreType.DMA,
            pltpu.SemaphoreType.DMA,
            pltpu.SemaphoreType.DMA,
            pltpu.SemaphoreType.DMA,
            pltpu.SemaphoreType.DMA,
            pltpu.SemaphoreType.DMA,
            pltpu.SemaphoreType.DMA,
            pltpu.SemaphoreType.DMA,
            pltpu.SemaphoreType.DMA,
            pltpu.SemaphoreType.DMA,
        ],
    )(h_pad, cl_pad, starts)


def kernel(x, cluster, edge_index, time_step_len,
           m0W1, m0b1, m0g1, m0e1, m0W2, m0b2, m0g2, m0e2,
           m1W1, m1b1, m1g1, m1e1, m1W2, m1b2, m1g2, m1e2,
           m2W1, m2b1, m2g1, m2e1, m2W2, m2b2, m2g2, m2e2,
           linW, linb):
    del edge_index, time_step_len

    cl_seg = jnp.pad(cluster, (0, N_PAD - N_NODES), constant_values=C_PAD)
    bounds = jnp.arange(NW + 1, dtype=jnp.int32) * CPT
    starts = jnp.sum((cluster[None, :] < bounds[:, None]).astype(jnp.int32),
                     axis=1)
    starts = jnp.pad(starts, (0, 48 - NW - 1), constant_values=N_NODES)

    h, a = _mlp0(x, m0W1, m0b1, m0g1, m0e1, m0W2, m0b2, m0g2, m0e2,
                 m1W1[:HID])

    g = _segmax_gather(h, cl_seg, starts)
    h, a = _mlp_cat(a, g, m1W1[HID:], m1b1, m1g1, m1e1, m1W2, m1b2, m1g2,
                    m1e2, m2W1[:HID])

    g = _segmax_gather(h, cl_seg, starts)
    hy = _mlp_cat_ext(a, g, m2W1[HID:], m2b1, m2g1, m2e1, m2W2, m2b2, m2g2,
                      m2e2, linW[:HID])

    agg2 = _segmax2(hy, cl_seg, starts)
    return _final(agg2, linW[HID:], linb)[:N_CLUSTERS]


# speedup vs baseline: 1.1037x; 1.1037x over previous
import functools

import jax
import jax.numpy as jnp
from jax import lax
from jax.experimental import pallas as pl
from jax.experimental.pallas import tpu as pltpu
from jax.experimental.pallas import tpu_sc as plsc

N_NODES = 50000
IN_CHS = 128
HID = 64
N_CLUSTERS = 2500

NC = 2
NS = 16
LANES = 16
NW = NC * NS

CPT = 80
C_PAD = NW * CPT
TC_TILE = 2000
TC_GRID = N_NODES // TC_TILE
N_PAD = 51200
SEG_CHUNK = 512
SEG_CHUNK2 = 256
G2 = 128


@functools.lru_cache(maxsize=None)
def _sc_mesh():
    return plsc.VectorSubcoreMesh(core_axis_name="c", subcore_axis_name="s")


def _ln_tail(hm, g, b):
    ones = jnp.full((HID, HID), 1.0 / HID, jnp.float32)
    d = hm[:, :HID] - hm[:, HID:]
    v = jnp.dot(d * d, ones)
    return jax.nn.relu(d * lax.rsqrt(v + 1e-5) * g + b)


def _hwide(h2):
    return jnp.concatenate(
        [h2, jnp.zeros((TC_TILE, HID), jnp.float32)], axis=1)


def _mlp0_body(x_ref, w1_ref, b1_ref, g1_ref, e1_ref, w2_ref, b2_ref,
               g2_ref, e2_ref, o_ref):
    hm = jnp.dot(x_ref[...], w1_ref[...]) + b1_ref[...]
    h = _ln_tail(hm, g1_ref[...], e1_ref[...])
    hm2 = jnp.dot(h, w2_ref[...]) + b2_ref[...]
    o_ref[...] = _hwide(_ln_tail(hm2, g2_ref[...], e2_ref[...]))


def _mlp_cat_body(h_ref, g_ref, w1a_ref, w1b_ref, b1_ref, g1_ref, e1_ref,
                  w2_ref, b2_ref, g2_ref, e2_ref, o_ref):
    hm = (jnp.dot(h_ref[...][:, :HID], w1a_ref[...])
          + jnp.dot(g_ref[...], w1b_ref[...]) + b1_ref[...])
    h = _ln_tail(hm, g1_ref[...], e1_ref[...])
    hm2 = jnp.dot(h, w2_ref[...]) + b2_ref[...]
    o_ref[...] = _hwide(_ln_tail(hm2, g2_ref[...], e2_ref[...]))


def _mlp_cat_ext_body(h_ref, g_ref, w1a_ref, w1b_ref, b1_ref, g1_ref, e1_ref,
                      w2_ref, b2_ref, g2_ref, e2_ref, wlin_ref, o_ref):
    hm = (jnp.dot(h_ref[...][:, :HID], w1a_ref[...])
          + jnp.dot(g_ref[...], w1b_ref[...]) + b1_ref[...])
    h = _ln_tail(hm, g1_ref[...], e1_ref[...])
    hm2 = jnp.dot(h, w2_ref[...]) + b2_ref[...]
    h2 = _ln_tail(hm2, g2_ref[...], e2_ref[...])
    y2 = jnp.dot(h2, wlin_ref[...])
    o_ref[...] = jnp.concatenate([h2, y2], axis=1)


def _final_body(a_ref, wb_ref, b_ref, o_ref):
    a = a_ref[...][:, :HID]
    y = a_ref[...][:, HID:]
    pre = y + jnp.dot(a, wb_ref[...]) + b_ref[...]
    n = jnp.sqrt(jnp.sum(pre * pre, axis=-1, keepdims=True))
    o_ref[...] = pre / jnp.maximum(n, 1e-12)


def _aug_w(W):
    m = jnp.broadcast_to(jnp.mean(W, axis=1, keepdims=True), W.shape)
    return jnp.concatenate([W, m], axis=1)


def _aug_b(b):
    return jnp.concatenate([b, jnp.broadcast_to(jnp.mean(b), b.shape)])


def _row_spec(width):
    return pl.BlockSpec((TC_TILE, width), lambda i: (i, 0))


def _full_spec(r, c):
    return pl.BlockSpec((r, c), lambda i: (0, 0))


def _vec_spec(n=HID):
    return pl.BlockSpec((n,), lambda i: (0,))


def _mlp0(x, w1, b1, g1, e1, w2, b2, g2, e2):
    return pl.pallas_call(
        _mlp0_body,
        grid=(TC_GRID,),
        in_specs=[_row_spec(IN_CHS), _full_spec(IN_CHS, 2 * HID),
                  _vec_spec(2 * HID), _vec_spec(), _vec_spec(),
                  _full_spec(HID, 2 * HID), _vec_spec(2 * HID), _vec_spec(),
                  _vec_spec()],
        out_specs=_row_spec(2 * HID),
        out_shape=jax.ShapeDtypeStruct((N_PAD, 2 * HID), jnp.float32),
    )(x, _aug_w(w1), _aug_b(b1), g1, e1, _aug_w(w2), _aug_b(b2), g2, e2)


def _mlp_cat(h, g, w1a, w1b, b1, g1, e1, w2, b2, g2, e2):
    return pl.pallas_call(
        _mlp_cat_body,
        grid=(TC_GRID,),
        in_specs=[_row_spec(2 * HID), _row_spec(HID),
                  _full_spec(HID, 2 * HID), _full_spec(HID, 2 * HID),
                  _vec_spec(2 * HID), _vec_spec(), _vec_spec(),
                  _full_spec(HID, 2 * HID), _vec_spec(2 * HID),
                  _vec_spec(), _vec_spec()],
        out_specs=_row_spec(2 * HID),
        out_shape=jax.ShapeDtypeStruct((N_PAD, 2 * HID), jnp.float32),
    )(h, g, _aug_w(w1a), _aug_w(w1b), _aug_b(b1), g1, e1, _aug_w(w2),
      _aug_b(b2), g2, e2)


def _mlp_cat_ext(h, g, w1a, w1b, b1, g1, e1, w2, b2, g2, e2, wlin):
    return pl.pallas_call(
        _mlp_cat_ext_body,
        grid=(TC_GRID,),
        in_specs=[_row_spec(2 * HID), _row_spec(HID),
                  _full_spec(HID, 2 * HID), _full_spec(HID, 2 * HID),
                  _vec_spec(2 * HID), _vec_spec(), _vec_spec(),
                  _full_spec(HID, 2 * HID), _vec_spec(2 * HID),
                  _vec_spec(), _vec_spec(), _full_spec(HID, HID)],
        out_specs=_row_spec(2 * HID),
        out_shape=jax.ShapeDtypeStruct((N_PAD, 2 * HID), jnp.float32),
    )(h, g, _aug_w(w1a), _aug_w(w1b), _aug_b(b1), g1, e1, _aug_w(w2),
      _aug_b(b2), g2, e2, wlin)


def _final(agg2, wb, b):
    return pl.pallas_call(
        _final_body,
        in_specs=[pl.BlockSpec((C_PAD, 2 * HID), lambda: (0, 0)),
                  pl.BlockSpec((HID, HID), lambda: (0, 0)),
                  pl.BlockSpec((HID,), lambda: (0,))],
        out_specs=pl.BlockSpec((C_PAD, HID), lambda: (0, 0)),
        out_shape=jax.ShapeDtypeStruct((C_PAD, HID), jnp.float32),
    )(agg2, wb, b)


def _tile_prologue(starts_hbm, stbuf, accbuf, chunk, width):
    wid = lax.axis_index("s") * NC + lax.axis_index("c")
    lane = lax.iota(jnp.int32, LANES)

    pltpu.sync_copy(starts_hbm, stbuf)

    def _extract(i):
        return plsc.load_gather(stbuf, [jnp.full((LANES,), i)])[0]

    s0 = _extract(wid)
    s1 = _extract(wid + 1)
    cbase = wid * CPT

    neg = jnp.full((LANES,), -jnp.inf, jnp.float32)
    kw = width // LANES

    def _init(i, carry):
        accbuf[i // kw, pl.ds((i % kw) * LANES, LANES)] = neg
        return carry

    lax.fori_loop(0, CPT * kw, _init, 0)

    base = (s0 // 8) * 8
    total = s1 - base
    nchunks = (total + chunk - 1) // chunk
    return wid, lane, cbase, base, nchunks, neg


def _segmax_pass(h_hbm, cl_hbm, hbuf, clbuf, accbuf, lane, cbase, base,
                 nchunks, neg, semh, semc, chunk, width):
    kw = width // LANES

    def _start(ci, b):
        st = base + ci * chunk
        pltpu.async_copy(h_hbm.at[pl.ds(st, chunk), pl.ds(0, width)],
                         hbuf.at[b], semh[b])
        pltpu.async_copy(cl_hbm.at[pl.ds(st, chunk)], clbuf.at[b], semc[b])

    def _wait(b):
        pltpu.make_async_copy(h_hbm.at[pl.ds(0, chunk), pl.ds(0, width)],
                              hbuf.at[b], semh[b]).wait()
        pltpu.make_async_copy(cl_hbm.at[pl.ds(0, chunk)], clbuf.at[b],
                              semc[b]).wait()

    def _compute(b, carry):
        def _group(gi, carry2):
            prev_cid, accs = carry2
            cids = clbuf[b, pl.ds(gi * LANES, LANES)]
            for k in range(LANES):
                j = gi * LANES + k
                cid = cids[k]
                c_loc = cid - cbase
                valid_v = jnp.full((LANES,), (c_loc >= 0) & (c_loc < CPT))
                same_v = jnp.full((LANES,), cid == prev_cid)
                row = jnp.full((LANES,), jnp.clip(c_loc, 0, CPT - 1))
                new_accs = []
                for w in range(kw):
                    r = hbuf[b, j, pl.ds(w * LANES, LANES)]
                    a = jnp.where(same_v, jnp.maximum(accs[w], r), r)
                    plsc.store_scatter(accbuf, [row, lane + w * LANES], a,
                                       mask=valid_v)
                    new_accs.append(a)
                accs = tuple(new_accs)
                prev_cid = cid
            return (prev_cid, accs)

        return lax.fori_loop(0, chunk // LANES, _group, carry)

    _start(0, 0)
    init = (jnp.int32(-1), (neg,) * kw)

    def _pair(pi, carry):
        for b in range(2):
            ci = pi * 2 + b

            def _proc(c, ci=ci, b=b):
                _start(ci + 1, 1 - b)
                _wait(b)
                return _compute(b, c)

            carry = lax.cond(ci < nchunks, _proc, lambda c: c, carry)
        return carry

    lax.fori_loop(0, (nchunks + 1) // 2, _pair, init)

    @pl.when(nchunks % 2 == 0)
    def _():
        _wait(0)

    @pl.when(nchunks % 2 == 1)
    def _():
        _wait(1)


def _segmax2_sc_body(h_hbm, cl_hbm, starts_hbm, agg_hbm, hbuf, clbuf, accbuf,
                     stbuf, semh0, semh1, semc0, semc1):
    wid, lane, cbase, base, nchunks, neg = _tile_prologue(
        starts_hbm, stbuf, accbuf, SEG_CHUNK2, 2 * HID)
    _segmax_pass(h_hbm, cl_hbm, hbuf, clbuf, accbuf, lane, cbase, base,
                 nchunks, neg, (semh0, semh1), (semc0, semc1), SEG_CHUNK2,
                 2 * HID)
    pltpu.sync_copy(accbuf, agg_hbm.at[pl.ds(cbase, CPT), :])


def _segmax2(h_pad, cl_pad, starts):
    return pl.kernel(
        _segmax2_sc_body,
        out_type=jax.ShapeDtypeStruct((C_PAD, 2 * HID), jnp.float32),
        mesh=_sc_mesh(),
        compiler_params=pltpu.CompilerParams(needs_layout_passes=False,
                                             use_tc_tiling_on_sc=False),
        scratch_types=[
            pltpu.VMEM((2, SEG_CHUNK2, 2 * HID), jnp.float32),
            pltpu.VMEM((2, SEG_CHUNK2), jnp.int32),
            pltpu.VMEM((CPT, 2 * HID), jnp.float32),
            pltpu.VMEM((48,), jnp.int32),
            pltpu.SemaphoreType.DMA,
            pltpu.SemaphoreType.DMA,
            pltpu.SemaphoreType.DMA,
            pltpu.SemaphoreType.DMA,
        ],
    )(h_pad, cl_pad, starts)


def _segmax_gather_sc_body(h_hbm, cl_hbm, starts_hbm, g_hbm, hbuf, clbuf,
                           accbuf, stbuf, cl2buf, idxbuf, gbuf,
                           semh0, semh1, semc0, semc1,
                           semi0, semi1, semo0, semo1):
    wid, lane, cbase, base, nchunks, neg = _tile_prologue(
        starts_hbm, stbuf, accbuf, SEG_CHUNK, HID)
    _segmax_pass(h_hbm, cl_hbm, hbuf, clbuf, accbuf, lane, cbase, base,
                 nchunks, neg, (semh0, semh1), (semc0, semc1), SEG_CHUNK, HID)

    semi = (semi0, semi1)
    semo = (semo0, semo1)
    dummy = jnp.int32(N_NODES) + wid
    nch2 = nchunks * (SEG_CHUNK // G2)

    def _start2(ci, b):
        st = base + ci * G2
        pltpu.async_copy(cl_hbm.at[pl.ds(st, G2)], cl2buf.at[b], semi[b])

    def _wait2(b):
        pltpu.make_async_copy(cl_hbm.at[pl.ds(0, G2)], cl2buf.at[b],
                              semi[b]).wait()

    def _wait_out(b):
        pltpu.make_async_copy(gbuf.at[b], g_hbm.at[pl.ds(0, G2), :],
                              semo[b]).wait()

    def _compute2(ci, b):
        start = base + ci * G2
        for grp in range(G2 // LANES):
            cids = cl2buf[b, pl.ds(grp * LANES, LANES)]
            c_loc = cids - cbase
            valid = (c_loc >= 0) & (c_loc < CPT)
            node_v = jnp.full((LANES,), start + grp * LANES) + lane
            idxbuf[b, pl.ds(grp * LANES, LANES)] = jnp.where(
                valid, node_v, jnp.full((LANES,), dummy))

        def _g2(gi, carry3):
            cids = cl2buf[b, pl.ds(gi * LANES, LANES)]
            for k in range(LANES):
                j = gi * LANES + k
                csp = jnp.full((LANES,),
                               jnp.clip(cids[k] - cbase, 0, CPT - 1))
                gbuf[b, j, pl.ds(0, LANES)] = plsc.load_gather(
                    accbuf, [csp, lane])
                gbuf[b, j, pl.ds(LANES, LANES)] = plsc.load_gather(
                    accbuf, [csp, lane + LANES])
                gbuf[b, j, pl.ds(2 * LANES, LANES)] = plsc.load_gather(
                    accbuf, [csp, lane + 2 * LANES])
                gbuf[b, j, pl.ds(3 * LANES, LANES)] = plsc.load_gather(
                    accbuf, [csp, lane + 3 * LANES])
            return carry3

        lax.fori_loop(0, G2 // LANES, _g2, 0)
        pltpu.async_copy(gbuf.at[b], g_hbm.at[idxbuf.at[b]], semo[b])

    _start2(0, 0)

    def _pair2(pi, carry):
        for b in range(2):
            ci = pi * 2 + b

            @pl.when(ci < nch2)
            def _(ci=ci, b=b):
                _start2(ci + 1, 1 - b)
                _wait2(b)

                @pl.when(ci >= 2)
                def _():
                    _wait_out(b)

                _compute2(ci, b)
        return carry

    lax.fori_loop(0, (nch2 + 1) // 2, _pair2, 0)

    @pl.when(nch2 % 2 == 0)
    def _():
        _wait2(0)

    @pl.when(nch2 % 2 == 1)
    def _():
        _wait2(1)

    @pl.when(nch2 >= 1)
    def _():
        _wait_out(0)

    @pl.when(nch2 >= 2)
    def _():
        _wait_out(1)


def _segmax_gather(h_pad, cl_pad, starts):
    return pl.kernel(
        _segmax_gather_sc_body,
        out_type=jax.ShapeDtypeStruct((N_PAD, HID), jnp.float32),
        mesh=_sc_mesh(),
        compiler_params=pltpu.CompilerParams(needs_layout_passes=False,
                                             use_tc_tiling_on_sc=False),
        scratch_types=[
            pltpu.VMEM((2, SEG_CHUNK, HID), jnp.float32),
            pltpu.VMEM((2, SEG_CHUNK), jnp.int32),
            pltpu.VMEM((CPT, HID), jnp.float32),
            pltpu.VMEM((48,), jnp.int32),
            pltpu.VMEM((2, G2), jnp.int32),
            pltpu.VMEM((2, G2), jnp.int32),
            pltpu.VMEM((2, G2, HID), jnp.float32),
            pltpu.SemaphoreType.DMA,
            pltpu.SemaphoreType.DMA,
            pltpu.SemaphoreType.DMA,
            pltpu.SemaphoreType.DMA,
            pltpu.SemaphoreType.DMA,
            pltpu.SemaphoreType.DMA,
            pltpu.SemaphoreType.DMA,
            pltpu.SemaphoreType.DMA,
        ],
    )(h_pad, cl_pad, starts)


def kernel(x, cluster, edge_index, time_step_len,
           m0W1, m0b1, m0g1, m0e1, m0W2, m0b2, m0g2, m0e2,
           m1W1, m1b1, m1g1, m1e1, m1W2, m1b2, m1g2, m1e2,
           m2W1, m2b1, m2g1, m2e1, m2W2, m2b2, m2g2, m2e2,
           linW, linb):
    del edge_index, time_step_len

    cl_seg = jnp.pad(cluster, (0, N_PAD - N_NODES), constant_values=C_PAD)
    bounds = jnp.arange(NW + 1, dtype=jnp.int32) * CPT
    starts = jnp.sum((cluster[None, :] < bounds[:, None]).astype(jnp.int32),
                     axis=1)
    starts = jnp.pad(starts, (0, 48 - NW - 1), constant_values=N_NODES)

    h = _mlp0(x, m0W1, m0b1, m0g1, m0e1, m0W2, m0b2, m0g2, m0e2)

    g = _segmax_gather(h, cl_seg, starts)
    h = _mlp_cat(h, g, m1W1[:HID], m1W1[HID:], m1b1, m1g1, m1e1, m1W2, m1b2,
                 m1g2, m1e2)

    g = _segmax_gather(h, cl_seg, starts)
    hy = _mlp_cat_ext(h, g, m2W1[:HID], m2W1[HID:], m2b1, m2g1, m2e1, m2W2,
                      m2b2, m2g2, m2e2, linW[:HID])

    agg2 = _segmax2(hy, cl_seg, starts)
    return _final(agg2, linW[HID:], linb)[:N_CLUSTERS]


# speedup vs baseline: 1.1352x; 1.0285x over previous
import functools

import jax
import jax.numpy as jnp
from jax import lax
from jax.experimental import pallas as pl
from jax.experimental.pallas import tpu as pltpu
from jax.experimental.pallas import tpu_sc as plsc

N_NODES = 50000
IN_CHS = 128
HID = 64
N_CLUSTERS = 2500

NC = 2
NS = 16
LANES = 16
NW = NC * NS

CPT = 80
C_PAD = NW * CPT
TC_TILE = 2000
TC_GRID = N_NODES // TC_TILE
N_PAD = 51200
SEG_CHUNK = 512
SEG_CHUNK2 = 256
G2 = 128


@functools.lru_cache(maxsize=None)
def _sc_mesh():
    return plsc.VectorSubcoreMesh(core_axis_name="c", subcore_axis_name="s")


def _ln_tail(hm, g, b):
    ones = jnp.full((HID, HID), 1.0 / HID, jnp.float32)
    d = hm[:, :HID] - hm[:, HID:]
    v = jnp.dot(d * d, ones)
    return jax.nn.relu(d * lax.rsqrt(v + 1e-5) * g + b)


def _hwide(h2):
    return jnp.concatenate(
        [h2, jnp.zeros((TC_TILE, HID), jnp.float32)], axis=1)


def _mlp0_body(x_ref, w1_ref, b1_ref, g1_ref, e1_ref, w2_ref, b2_ref,
               g2_ref, e2_ref, o_ref):
    hm = jnp.dot(x_ref[...], w1_ref[...]) + b1_ref[...]
    h = _ln_tail(hm, g1_ref[...], e1_ref[...])
    hm2 = jnp.dot(h, w2_ref[...]) + b2_ref[...]
    o_ref[...] = _hwide(_ln_tail(hm2, g2_ref[...], e2_ref[...]))


def _mlp_cat_body(h_ref, g_ref, w1a_ref, w1b_ref, b1_ref, g1_ref, e1_ref,
                  w2_ref, b2_ref, g2_ref, e2_ref, o_ref):
    hm = (jnp.dot(h_ref[...][:, :HID], w1a_ref[...])
          + jnp.dot(g_ref[...][:, :HID], w1b_ref[...]) + b1_ref[...])
    h = _ln_tail(hm, g1_ref[...], e1_ref[...])
    hm2 = jnp.dot(h, w2_ref[...]) + b2_ref[...]
    o_ref[...] = _hwide(_ln_tail(hm2, g2_ref[...], e2_ref[...]))


def _mlp_cat_ext_body(h_ref, g_ref, w1a_ref, w1b_ref, b1_ref, g1_ref, e1_ref,
                      w2_ref, b2_ref, g2_ref, e2_ref, wlin_ref, o_ref):
    hm = (jnp.dot(h_ref[...][:, :HID], w1a_ref[...])
          + jnp.dot(g_ref[...][:, :HID], w1b_ref[...]) + b1_ref[...])
    h = _ln_tail(hm, g1_ref[...], e1_ref[...])
    hm2 = jnp.dot(h, w2_ref[...]) + b2_ref[...]
    h2 = _ln_tail(hm2, g2_ref[...], e2_ref[...])
    y2 = jnp.dot(h2, wlin_ref[...])
    o_ref[...] = jnp.concatenate([h2, y2], axis=1)


def _final_body(a_ref, wb_ref, b_ref, o_ref):
    a = a_ref[...][:, :HID]
    y = a_ref[...][:, HID:]
    pre = y + jnp.dot(a, wb_ref[...]) + b_ref[...]
    n = jnp.sqrt(jnp.sum(pre * pre, axis=-1, keepdims=True))
    o_ref[...] = pre / jnp.maximum(n, 1e-12)


def _aug_w(W):
    m = jnp.broadcast_to(jnp.mean(W, axis=1, keepdims=True), W.shape)
    return jnp.concatenate([W, m], axis=1)


def _aug_b(b):
    return jnp.concatenate([b, jnp.broadcast_to(jnp.mean(b), b.shape)])


def _row_spec(width):
    return pl.BlockSpec((TC_TILE, width), lambda i: (i, 0))


def _full_spec(r, c):
    return pl.BlockSpec((r, c), lambda i: (0, 0))


def _vec_spec(n=HID):
    return pl.BlockSpec((n,), lambda i: (0,))


def _mlp0(x, w1, b1, g1, e1, w2, b2, g2, e2):
    return pl.pallas_call(
        _mlp0_body,
        grid=(TC_GRID,),
        in_specs=[_row_spec(IN_CHS), _full_spec(IN_CHS, 2 * HID),
                  _vec_spec(2 * HID), _vec_spec(), _vec_spec(),
                  _full_spec(HID, 2 * HID), _vec_spec(2 * HID), _vec_spec(),
                  _vec_spec()],
        out_specs=_row_spec(2 * HID),
        out_shape=jax.ShapeDtypeStruct((N_PAD, 2 * HID), jnp.float32),
    )(x, _aug_w(w1), _aug_b(b1), g1, e1, _aug_w(w2), _aug_b(b2), g2, e2)


def _mlp_cat(h, g, w1a, w1b, b1, g1, e1, w2, b2, g2, e2):
    return pl.pallas_call(
        _mlp_cat_body,
        grid=(TC_GRID,),
        in_specs=[_row_spec(2 * HID), _row_spec(2 * HID),
                  _full_spec(HID, 2 * HID), _full_spec(HID, 2 * HID),
                  _vec_spec(2 * HID), _vec_spec(), _vec_spec(),
                  _full_spec(HID, 2 * HID), _vec_spec(2 * HID),
                  _vec_spec(), _vec_spec()],
        out_specs=_row_spec(2 * HID),
        out_shape=jax.ShapeDtypeStruct((N_PAD, 2 * HID), jnp.float32),
    )(h, g, _aug_w(w1a), _aug_w(w1b), _aug_b(b1), g1, e1, _aug_w(w2),
      _aug_b(b2), g2, e2)


def _mlp_cat_ext(h, g, w1a, w1b, b1, g1, e1, w2, b2, g2, e2, wlin):
    return pl.pallas_call(
        _mlp_cat_ext_body,
        grid=(TC_GRID,),
        in_specs=[_row_spec(2 * HID), _row_spec(2 * HID),
                  _full_spec(HID, 2 * HID), _full_spec(HID, 2 * HID),
                  _vec_spec(2 * HID), _vec_spec(), _vec_spec(),
                  _full_spec(HID, 2 * HID), _vec_spec(2 * HID),
                  _vec_spec(), _vec_spec(), _full_spec(HID, HID)],
        out_specs=_row_spec(2 * HID),
        out_shape=jax.ShapeDtypeStruct((N_PAD, 2 * HID), jnp.float32),
    )(h, g, _aug_w(w1a), _aug_w(w1b), _aug_b(b1), g1, e1, _aug_w(w2),
      _aug_b(b2), g2, e2, wlin)


def _final(agg2, wb, b):
    return pl.pallas_call(
        _final_body,
        in_specs=[pl.BlockSpec((C_PAD, 2 * HID), lambda: (0, 0)),
                  pl.BlockSpec((HID, HID), lambda: (0, 0)),
                  pl.BlockSpec((HID,), lambda: (0,))],
        out_specs=pl.BlockSpec((C_PAD, HID), lambda: (0, 0)),
        out_shape=jax.ShapeDtypeStruct((C_PAD, HID), jnp.float32),
    )(agg2, wb, b)


def _tile_prologue(starts_hbm, stbuf, accbuf, chunk, width):
    wid = lax.axis_index("s") * NC + lax.axis_index("c")
    lane = lax.iota(jnp.int32, LANES)

    pltpu.sync_copy(starts_hbm, stbuf)

    def _extract(i):
        return plsc.load_gather(stbuf, [jnp.full((LANES,), i)])[0]

    s0 = _extract(wid)
    s1 = _extract(wid + 1)
    cbase = wid * CPT

    neg = jnp.full((LANES,), -jnp.inf, jnp.float32)
    kw = width // LANES

    def _init(i, carry):
        accbuf[i // kw, pl.ds((i % kw) * LANES, LANES)] = neg
        return carry

    lax.fori_loop(0, CPT * kw, _init, 0)

    base = (s0 // 8) * 8
    total = s1 - base
    nchunks = (total + chunk - 1) // chunk
    return wid, lane, cbase, base, nchunks, neg, s0, s1


def _segmax_pass(h_hbm, cl_hbm, hbuf, clbuf, accbuf, lane, cbase, base,
                 nchunks, neg, semh, semc, chunk, width):
    kw = width // LANES

    def _start(ci, b):
        st = base + ci * chunk
        pltpu.async_copy(h_hbm.at[pl.ds(st, chunk), pl.ds(0, width)],
                         hbuf.at[b], semh[b])
        pltpu.async_copy(cl_hbm.at[pl.ds(st, chunk)], clbuf.at[b], semc[b])

    def _wait(b):
        pltpu.make_async_copy(h_hbm.at[pl.ds(0, chunk), pl.ds(0, width)],
                              hbuf.at[b], semh[b]).wait()
        pltpu.make_async_copy(cl_hbm.at[pl.ds(0, chunk)], clbuf.at[b],
                              semc[b]).wait()

    def _compute(b, carry):
        def _group(gi, carry2):
            prev_cid, accs = carry2
            cids = clbuf[b, pl.ds(gi * LANES, LANES)]
            for k in range(LANES):
                j = gi * LANES + k
                cid = cids[k]
                c_loc = cid - cbase
                valid_v = jnp.full((LANES,), (c_loc >= 0) & (c_loc < CPT))
                same_v = jnp.full((LANES,), cid == prev_cid)
                row = jnp.full((LANES,), jnp.clip(c_loc, 0, CPT - 1))
                new_accs = []
                for w in range(kw):
                    r = hbuf[b, j, pl.ds(w * LANES, LANES)]
                    a = jnp.where(same_v, jnp.maximum(accs[w], r), r)
                    plsc.store_scatter(accbuf, [row, lane + w * LANES], a,
                                       mask=valid_v)
                    new_accs.append(a)
                accs = tuple(new_accs)
                prev_cid = cid
            return (prev_cid, accs)

        return lax.fori_loop(0, chunk // LANES, _group, carry)

    _start(0, 0)
    init = (jnp.int32(-1), (neg,) * kw)

    def _pair(pi, carry):
        for b in range(2):
            ci = pi * 2 + b

            def _proc(c, ci=ci, b=b):
                _start(ci + 1, 1 - b)
                _wait(b)
                return _compute(b, c)

            carry = lax.cond(ci < nchunks, _proc, lambda c: c, carry)
        return carry

    lax.fori_loop(0, (nchunks + 1) // 2, _pair, init)

    @pl.when(nchunks % 2 == 0)
    def _():
        _wait(0)

    @pl.when(nchunks % 2 == 1)
    def _():
        _wait(1)


def _segmax2_sc_body(h_hbm, cl_hbm, starts_hbm, agg_hbm, hbuf, clbuf, accbuf,
                     stbuf, semh0, semh1, semc0, semc1):
    wid, lane, cbase, base, nchunks, neg, s0, s1 = _tile_prologue(
        starts_hbm, stbuf, accbuf, SEG_CHUNK2, 2 * HID)
    _segmax_pass(h_hbm, cl_hbm, hbuf, clbuf, accbuf, lane, cbase, base,
                 nchunks, neg, (semh0, semh1), (semc0, semc1), SEG_CHUNK2,
                 2 * HID)
    pltpu.sync_copy(accbuf, agg_hbm.at[pl.ds(cbase, CPT), :])


def _segmax2(h_pad, cl_pad, starts):
    return pl.kernel(
        _segmax2_sc_body,
        out_type=jax.ShapeDtypeStruct((C_PAD, 2 * HID), jnp.float32),
        mesh=_sc_mesh(),
        compiler_params=pltpu.CompilerParams(needs_layout_passes=False,
                                             use_tc_tiling_on_sc=False),
        scratch_types=[
            pltpu.VMEM((2, SEG_CHUNK2, 2 * HID), jnp.float32),
            pltpu.VMEM((2, SEG_CHUNK2), jnp.int32),
            pltpu.VMEM((CPT, 2 * HID), jnp.float32),
            pltpu.VMEM((48,), jnp.int32),
            pltpu.SemaphoreType.DMA,
            pltpu.SemaphoreType.DMA,
            pltpu.SemaphoreType.DMA,
            pltpu.SemaphoreType.DMA,
        ],
    )(h_pad, cl_pad, starts)


def _segmax_gather_sc_body(h_hbm, cl_hbm, starts_hbm, g_hbm, hbuf, clbuf,
                           accbuf, stbuf, cl2buf, idxbuf, gbuf,
                           semh0, semh1, semc0, semc1,
                           semi0, semi1, semo0, semo1, semb):
    wid, lane, cbase, base, nchunks, neg, s0, s1 = _tile_prologue(
        starts_hbm, stbuf, accbuf, SEG_CHUNK, HID)
    _segmax_pass(h_hbm, cl_hbm, hbuf, clbuf, accbuf, lane, cbase, base,
                 nchunks, neg, (semh0, semh1), (semc0, semc1), SEG_CHUNK, HID)

    semi = (semi0, semi1)
    semo = (semo0, semo1)
    dummy = jnp.int32(N_NODES) + wid
    nch2 = nchunks * (SEG_CHUNK // G2)

    def _start2(ci, b):
        st = base + ci * G2
        pltpu.async_copy(cl_hbm.at[pl.ds(st, G2)], cl2buf.at[b], semi[b])

    def _wait2(b):
        pltpu.make_async_copy(cl_hbm.at[pl.ds(0, G2)], cl2buf.at[b],
                              semi[b]).wait()

    def _wait_out(b):
        pltpu.make_async_copy(gbuf.at[b].at[:, pl.ds(0, HID)],
                              g_hbm.at[pl.ds(0, G2), pl.ds(0, HID)],
                              semo[b]).wait()

    def _interior(ci):
        st = base + ci * G2
        return (st >= s0) & (st + G2 <= s1)

    zv = jnp.zeros((LANES,), jnp.float32)
    for bb in range(2):
        def _zf(i, c, bb=bb):
            for w in range(4, 8):
                gbuf[bb, i, pl.ds(w * LANES, LANES)] = zv
            return c

        lax.fori_loop(0, G2, _zf, 0)

    def _compute2(ci, b):
        start = base + ci * G2
        for grp in range(G2 // LANES):
            cids = cl2buf[b, pl.ds(grp * LANES, LANES)]
            c_loc = cids - cbase
            valid = (c_loc >= 0) & (c_loc < CPT)
            node_v = jnp.full((LANES,), start + grp * LANES) + lane
            idxbuf[b, pl.ds(grp * LANES, LANES)] = jnp.where(
                valid, node_v, jnp.full((LANES,), dummy))

        def _g2(gi, carry3):
            cids = cl2buf[b, pl.ds(gi * LANES, LANES)]
            for k in range(LANES):
                j = gi * LANES + k
                csp = jnp.full((LANES,),
                               jnp.clip(cids[k] - cbase, 0, CPT - 1))
                gbuf[b, j, pl.ds(0, LANES)] = plsc.load_gather(
                    accbuf, [csp, lane])
                gbuf[b, j, pl.ds(LANES, LANES)] = plsc.load_gather(
                    accbuf, [csp, lane + LANES])
                gbuf[b, j, pl.ds(2 * LANES, LANES)] = plsc.load_gather(
                    accbuf, [csp, lane + 2 * LANES])
                gbuf[b, j, pl.ds(3 * LANES, LANES)] = plsc.load_gather(
                    accbuf, [csp, lane + 3 * LANES])
            return carry3

        lax.fori_loop(0, G2 // LANES, _g2, 0)

        @pl.when(_interior(ci))
        def _():
            pltpu.async_copy(gbuf.at[b].at[:, pl.ds(0, HID)],
                             g_hbm.at[pl.ds(start, G2), pl.ds(0, HID)],
                             semo[b])

        @pl.when(jnp.logical_not(_interior(ci)))
        def _():
            pltpu.async_copy(gbuf.at[b], g_hbm.at[idxbuf.at[b]],
                             semb).wait()

    _start2(0, 0)

    def _pair2(pi, carry):
        for b in range(2):
            ci = pi * 2 + b

            @pl.when(ci < nch2)
            def _(ci=ci, b=b):
                _start2(ci + 1, 1 - b)
                _wait2(b)

                @pl.when((ci >= 2) & _interior(ci - 2))
                def _():
                    _wait_out(b)

                _compute2(ci, b)
        return carry

    lax.fori_loop(0, (nch2 + 1) // 2, _pair2, 0)

    @pl.when(nch2 % 2 == 0)
    def _():
        _wait2(0)

    @pl.when(nch2 % 2 == 1)
    def _():
        _wait2(1)

    for bb in range(2):
        lc = nch2 - 1 - ((nch2 - 1 - bb) % 2)

        @pl.when((lc >= 0) & _interior(lc))
        def _(bb=bb):
            _wait_out(bb)


def _segmax_gather(h_pad, cl_pad, starts):
    return pl.kernel(
        _segmax_gather_sc_body,
        out_type=jax.ShapeDtypeStruct((N_PAD, 2 * HID), jnp.float32),
        mesh=_sc_mesh(),
        compiler_params=pltpu.CompilerParams(needs_layout_passes=False,
                                             use_tc_tiling_on_sc=False),
        scratch_types=[
            pltpu.VMEM((2, SEG_CHUNK, HID), jnp.float32),
            pltpu.VMEM((2, SEG_CHUNK), jnp.int32),
            pltpu.VMEM((CPT, HID), jnp.float32),
            pltpu.VMEM((48,), jnp.int32),
            pltpu.VMEM((2, G2), jnp.int32),
            pltpu.VMEM((2, G2), jnp.int32),
            pltpu.VMEM((2, G2, 2 * HID), jnp.float32),
            pltpu.SemaphoreType.DMA,
            pltpu.SemaphoreType.DMA,
            pltpu.SemaphoreType.DMA,
            pltpu.SemaphoreType.DMA,
            pltpu.SemaphoreType.DMA,
            pltpu.SemaphoreType.DMA,
            pltpu.SemaphoreType.DMA,
            pltpu.SemaphoreType.DMA,
            pltpu.SemaphoreType.DMA,
        ],
    )(h_pad, cl_pad, starts)


def kernel(x, cluster, edge_index, time_step_len,
           m0W1, m0b1, m0g1, m0e1, m0W2, m0b2, m0g2, m0e2,
           m1W1, m1b1, m1g1, m1e1, m1W2, m1b2, m1g2, m1e2,
           m2W1, m2b1, m2g1, m2e1, m2W2, m2b2, m2g2, m2e2,
           linW, linb):
    del edge_index, time_step_len

    cl_seg = jnp.pad(cluster, (0, N_PAD - N_NODES), constant_values=C_PAD)
    bounds = jnp.arange(NW + 1, dtype=jnp.int32) * CPT
    starts = jnp.sum((cluster[None, :] < bounds[:, None]).astype(jnp.int32),
                     axis=1)
    starts = jnp.pad(starts, (0, 48 - NW - 1), constant_values=N_NODES)

    h = _mlp0(x, m0W1, m0b1, m0g1, m0e1, m0W2, m0b2, m0g2, m0e2)

    g = _segmax_gather(h, cl_seg, starts)
    h = _mlp_cat(h, g, m1W1[:HID], m1W1[HID:], m1b1, m1g1, m1e1, m1W2, m1b2,
                 m1g2, m1e2)

    g = _segmax_gather(h, cl_seg, starts)
    hy = _mlp_cat_ext(h, g, m2W1[:HID], m2W1[HID:], m2b1, m2g1, m2e1, m2W2,
                      m2b2, m2g2, m2e2, linW[:HID])

    agg2 = _segmax2(hy, cl_seg, starts)
    return _final(agg2, linW[HID:], linb)[:N_CLUSTERS]
